# baseline (speedup 1.0000x reference)
import functools
import math

import jax
import jax.numpy as jnp
import numpy as np
from jax import lax
from jax.experimental import pallas as pl
from jax.experimental.pallas import tpu as pltpu
from jax.experimental.pallas import tpu_sc as plsc

V = 16384
D = 768
H = 12
DFF = 1024
E = 8
TOPK = 2
S = 2048
DK = D // H

_NEG = -1e9


def _pe_np(seq_len, d):
    position = np.arange(seq_len, dtype=np.float32)[:, None]
    div_term = np.exp(np.arange(0, d, 2, dtype=np.float32) * (-math.log(10000.0) / d))
    pe = np.zeros((seq_len, d), dtype=np.float32)
    pe[:, 0::2] = np.sin(position * div_term)
    pe[:, 1::2] = np.cos(position * div_term)
    return pe


def _sc_embed_gather(table, ids, n_rows):
    nw = 32
    per_w = n_rows // nw
    mesh = plsc.VectorSubcoreMesh(core_axis_name="c", subcore_axis_name="s")

    @functools.partial(
        pl.kernel,
        mesh=mesh,
        out_type=jax.ShapeDtypeStruct((n_rows, D), jnp.float32),
        scratch_types=[
            pltpu.VMEM((per_w,), jnp.int32),
            pltpu.VMEM((per_w, D), jnp.float32),
            pltpu.SemaphoreType.DMA,
        ],
    )
    def k(table_hbm, ids_hbm, out_hbm, idx_v, rows_v, sem):
        wid = lax.axis_index("s") * 2 + lax.axis_index("c")
        base = wid * per_w
        pltpu.sync_copy(ids_hbm.at[pl.ds(base, per_w)], idx_v)
        pltpu.async_copy(table_hbm.at[idx_v], rows_v, sem).wait()
        pltpu.sync_copy(rows_v, out_hbm.at[pl.ds(base, per_w)])

    return k(table, ids)


def _mm_kernel(x_ref, w_ref, b_ref, o_ref):
    acc = lax.dot_general(
        x_ref[...], w_ref[...], (((1,), (0,)), ((), ())),
        preferred_element_type=jnp.float32)
    o_ref[...] = acc + b_ref[...]


def _mm_bias(x, w, b, bn=256):
    m, k = x.shape
    _, n = w.shape
    return pl.pallas_call(
        _mm_kernel,
        grid=(n // bn,),
        in_specs=[
            pl.BlockSpec((m, k), lambda j: (0, 0)),
            pl.BlockSpec((k, bn), lambda j: (0, j)),
            pl.BlockSpec((1, bn), lambda j: (0, j)),
        ],
        out_specs=pl.BlockSpec((m, bn), lambda j: (0, j)),
        out_shape=jax.ShapeDtypeStruct((m, n), jnp.float32),
        compiler_params=pltpu.CompilerParams(
            dimension_semantics=("arbitrary",)),
    )(x, w, b.reshape(1, n))


def _add_kernel(a_ref, b_ref, o_ref):
    o_ref[...] = a_ref[...] + b_ref[...]


def _add(a, b, bm=256):
    m, n = a.shape
    return pl.pallas_call(
        _add_kernel,
        grid=(m // bm,),
        in_specs=[
            pl.BlockSpec((bm, n), lambda i: (i, 0)),
            pl.BlockSpec((bm, n), lambda i: (i, 0)),
        ],
        out_specs=pl.BlockSpec((bm, n), lambda i: (i, 0)),
        out_shape=jax.ShapeDtypeStruct((m, n), jnp.float32),
    )(a, b)


def _ln_kernel(x_ref, r_ref, g_ref, b_ref, o_ref):
    x = x_ref[...] + r_ref[...]
    mu = jnp.mean(x, axis=-1, keepdims=True)
    xc = x - mu
    var = jnp.mean(xc * xc, axis=-1, keepdims=True)
    o_ref[...] = xc * lax.rsqrt(var + 1e-5) * g_ref[...] + b_ref[...]


def _ln_res(x, r, g, b, bm=256):
    m, n = x.shape
    return pl.pallas_call(
        _ln_kernel,
        grid=(m // bm,),
        in_specs=[
            pl.BlockSpec((bm, n), lambda i: (i, 0)),
            pl.BlockSpec((bm, n), lambda i: (i, 0)),
            pl.BlockSpec((1, n), lambda i: (0, 0)),
            pl.BlockSpec((1, n), lambda i: (0, 0)),
        ],
        out_specs=pl.BlockSpec((bm, n), lambda i: (i, 0)),
        out_shape=jax.ShapeDtypeStruct((m, n), jnp.float32),
    )(x, r, g.reshape(1, n), b.reshape(1, n))


def _attn_kernel(q_ref, k_ref, v_ref, o_ref, *, causal, bq):
    i = pl.program_id(0)
    if causal:
        rows = i * bq + lax.broadcasted_iota(jnp.int32, (bq, k_ref.shape[0]), 0)
        cols = lax.broadcasted_iota(jnp.int32, (bq, k_ref.shape[0]), 1)
        mask = rows >= cols
    for h in range(H):
        sl = pl.ds(h * DK, DK)
        q = q_ref[:, sl].astype(jnp.bfloat16)
        k = k_ref[:, sl].astype(jnp.bfloat16)
        v = v_ref[:, sl].astype(jnp.bfloat16)
        s = lax.dot_general(q, k, (((1,), (1,)), ((), ())),
                            preferred_element_type=jnp.float32)
        s = s * (1.0 / math.sqrt(DK))
        if causal:
            s = jnp.where(mask, s, _NEG)
        m = jnp.max(s, axis=-1, keepdims=True)
        p = jnp.exp(s - m)
        p = (p / jnp.sum(p, axis=-1, keepdims=True)).astype(jnp.bfloat16)
        o_ref[:, sl] = lax.dot_general(p, v, (((1,), (0,)), ((), ())),
                                       preferred_element_type=jnp.float32)


def _attention(q2, k2, v2, causal, bq=256):
    sq, d = q2.shape
    skv = k2.shape[0]
    return pl.pallas_call(
        functools.partial(_attn_kernel, causal=causal, bq=bq),
        grid=(sq // bq,),
        in_specs=[
            pl.BlockSpec((bq, d), lambda i: (i, 0)),
            pl.BlockSpec((skv, d), lambda i: (0, 0)),
            pl.BlockSpec((skv, d), lambda i: (0, 0)),
        ],
        out_specs=pl.BlockSpec((bq, d), lambda i: (i, 0)),
        out_shape=jax.ShapeDtypeStruct((sq, d), jnp.float32),
        compiler_params=pltpu.CompilerParams(
            dimension_semantics=("arbitrary",)),
    )(q2, k2, v2)


def _router_kernel(x_ref, wr_ref, br_ref, g_ref, c_ref):
    logits = lax.dot_general(x_ref[...], wr_ref[...], (((1,), (0,)), ((), ())),
                             preferred_element_type=jnp.float32) + br_ref[...]
    mx = jnp.max(logits, axis=-1, keepdims=True)
    ex = jnp.exp(logits - mx)
    probs = ex / jnp.sum(ex, axis=-1, keepdims=True)
    iota = lax.broadcasted_iota(jnp.int32, probs.shape, 1)
    m1 = jnp.max(probs, axis=-1, keepdims=True)
    i1 = jnp.min(jnp.where(probs == m1, iota, E), axis=-1, keepdims=True)
    oh1 = iota == i1
    p2 = jnp.where(oh1, -1.0, probs)
    m2 = jnp.max(p2, axis=-1, keepdims=True)
    i2 = jnp.min(jnp.where(p2 == m2, iota, E), axis=-1, keepdims=True)
    oh2 = iota == i2
    denom = m1 + m2
    g_ref[...] = (jnp.where(oh1, m1, 0.0) + jnp.where(oh2, m2, 0.0)) / denom

    blockcnt = jnp.sum(jnp.where(oh1 | oh2, 1.0, 0.0), axis=0, keepdims=True)

    @pl.when(pl.program_id(0) == 0)
    def _():
        c_ref[...] = jnp.zeros_like(c_ref)

    c_ref[...] += blockcnt


def _router(x, wr, br, bm=256):
    m, k = x.shape
    e = wr.shape[1]
    return pl.pallas_call(
        _router_kernel,
        grid=(m // bm,),
        in_specs=[
            pl.BlockSpec((bm, k), lambda i: (i, 0)),
            pl.BlockSpec((k, e), lambda i: (0, 0)),
            pl.BlockSpec((1, e), lambda i: (0, 0)),
        ],
        out_specs=[
            pl.BlockSpec((bm, e), lambda i: (i, 0)),
            pl.BlockSpec((1, e), lambda i: (0, 0)),
        ],
        out_shape=[
            jax.ShapeDtypeStruct((m, e), jnp.float32),
            jax.ShapeDtypeStruct((1, e), jnp.float32),
        ],
        compiler_params=pltpu.CompilerParams(
            dimension_semantics=("arbitrary",)),
    )(x, wr, br.reshape(1, e))


def _plan_kernel(gf_ref, gb_ref, s_ref, *, bm, s_tot):
    i = pl.program_id(0)
    m = jnp.where(gf_ref[...] > 0.0, 1.0, 0.0)
    rows = i * bm + lax.broadcasted_iota(jnp.int32, (bm, s_tot), 0)
    cols = lax.broadcasted_iota(jnp.int32, (bm, s_tot), 1)
    lt = jnp.where(cols < rows, 1.0, 0.0)
    cnt = lax.dot_general(lt, m, (((1,), (0,)), ((), ())),
                          preferred_element_type=jnp.float32)
    g = gb_ref[...]
    iota = lax.broadcasted_iota(jnp.int32, g.shape, 1)
    sel = g > 0.0
    e1 = jnp.min(jnp.where(sel, iota, E), axis=-1, keepdims=True)
    e1 = jnp.minimum(e1, E - 1)
    e2 = jnp.max(jnp.where(sel, iota, 0), axis=-1, keepdims=True)
    rank1 = jnp.sum(jnp.where(iota == e1, cnt, 0.0), axis=-1, keepdims=True)
    rank2 = jnp.sum(jnp.where(iota == e2, cnt, 0.0), axis=-1, keepdims=True)
    slot1 = e1 * s_tot + rank1.astype(jnp.int32)
    slot2 = e2 * s_tot + rank2.astype(jnp.int32)
    s_ref[...] = (jnp.where(iota == 0, slot1, 0)
                  + jnp.where(iota == 1, slot2, 0))


def _route_plan(gates, bm=256):
    m, e = gates.shape
    return pl.pallas_call(
        functools.partial(_plan_kernel, bm=bm, s_tot=m),
        grid=(m // bm,),
        in_specs=[
            pl.BlockSpec((m, e), lambda i: (0, 0)),
            pl.BlockSpec((bm, e), lambda i: (i, 0)),
        ],
        out_specs=pl.BlockSpec((bm, e), lambda i: (i, 0)),
        out_shape=jax.ShapeDtypeStruct((m, e), jnp.int32),
    )(gates, gates)


def _sc_dispatch(x, slot1, slot2):
    nw = 32
    per_w = S // nw
    mesh = plsc.VectorSubcoreMesh(core_axis_name="c", subcore_axis_name="s")

    @functools.partial(
        pl.kernel,
        mesh=mesh,
        out_type=jax.ShapeDtypeStruct((E * S, D), jnp.float32),
        scratch_types=[
            pltpu.VMEM((per_w,), jnp.int32),
            pltpu.VMEM((per_w,), jnp.int32),
            pltpu.VMEM((per_w, D), jnp.float32),
            pltpu.SemaphoreType.DMA,
            pltpu.SemaphoreType.DMA,
        ],
    )
    def k(x_hbm, s1_hbm, s2_hbm, xg_hbm, i1_v, i2_v, rows_v, sem1, sem2):
        wid = lax.axis_index("s") * 2 + lax.axis_index("c")
        base = wid * per_w
        pltpu.sync_copy(x_hbm.at[pl.ds(base, per_w)], rows_v)
        pltpu.sync_copy(s1_hbm.at[pl.ds(base, per_w)], i1_v)
        pltpu.sync_copy(s2_hbm.at[pl.ds(base, per_w)], i2_v)
        c1 = pltpu.async_copy(rows_v, xg_hbm.at[i1_v], sem1)
        c2 = pltpu.async_copy(rows_v, xg_hbm.at[i2_v], sem2)
        c1.wait()
        c2.wait()

    return k(x, slot1, slot2)


def _sc_combine_gather(yg, slot1, slot2):
    nw = 32
    per_w = S // nw
    mesh = plsc.VectorSubcoreMesh(core_axis_name="c", subcore_axis_name="s")

    @functools.partial(
        pl.kernel,
        mesh=mesh,
        out_type=[
            jax.ShapeDtypeStruct((S, D), jnp.float32),
            jax.ShapeDtypeStruct((S, D), jnp.float32),
        ],
        scratch_types=[
            pltpu.VMEM((per_w,), jnp.int32),
            pltpu.VMEM((per_w,), jnp.int32),
            pltpu.VMEM((per_w, D), jnp.float32),
            pltpu.VMEM((per_w, D), jnp.float32),
            pltpu.SemaphoreType.DMA,
            pltpu.SemaphoreType.DMA,
        ],
    )
    def k(yg_hbm, s1_hbm, s2_hbm, y1_hbm, y2_hbm,
          i1_v, i2_v, r1_v, r2_v, sem1, sem2):
        wid = lax.axis_index("s") * 2 + lax.axis_index("c")
        base = wid * per_w
        pltpu.sync_copy(s1_hbm.at[pl.ds(base, per_w)], i1_v)
        pltpu.sync_copy(s2_hbm.at[pl.ds(base, per_w)], i2_v)
        c1 = pltpu.async_copy(yg_hbm.at[i1_v], r1_v, sem1)
        c2 = pltpu.async_copy(yg_hbm.at[i2_v], r2_v, sem2)
        c1.wait()
        c2.wait()
        pltpu.sync_copy(r1_v, y1_hbm.at[pl.ds(base, per_w)])
        pltpu.sync_copy(r2_v, y2_hbm.at[pl.ds(base, per_w)])

    return k(yg, slot1, slot2)


def _ffn_kernel(cnt_ref, xg_ref, w1_ref, b1_ref, w2_ref, b2_ref, o_ref, *, bm):
    e = pl.program_id(0)
    j = pl.program_id(1)

    @pl.when(j * bm < cnt_ref[e])
    def _():
        h1 = lax.dot_general(xg_ref[0], w1_ref[0], (((1,), (0,)), ((), ())),
                             preferred_element_type=jnp.float32) + b1_ref[0]
        h1 = jnp.maximum(h1, 0.0)
        o_ref[0] = lax.dot_general(h1, w2_ref[0], (((1,), (0,)), ((), ())),
                                   preferred_element_type=jnp.float32) + b2_ref[0]


def _ffn_sparse(xg3, counts, w1, b1, w2, b2, bm=256):
    e, s, d = xg3.shape
    dff = w1.shape[2]

    def xg_map(ei, j, cnt):
        nb = jnp.maximum(pl.cdiv(cnt[ei], bm) - 1, 0)
        return (ei, jnp.minimum(j, nb), 0)

    grid_spec = pltpu.PrefetchScalarGridSpec(
        num_scalar_prefetch=1,
        grid=(e, s // bm),
        in_specs=[
            pl.BlockSpec((1, bm, d), xg_map),
            pl.BlockSpec((1, d, dff), lambda ei, j, cnt: (ei, 0, 0)),
            pl.BlockSpec((1, 1, dff), lambda ei, j, cnt: (ei, 0, 0)),
            pl.BlockSpec((1, dff, d), lambda ei, j, cnt: (ei, 0, 0)),
            pl.BlockSpec((1, 1, d), lambda ei, j, cnt: (ei, 0, 0)),
        ],
        out_specs=pl.BlockSpec((1, bm, d), xg_map),
    )
    return pl.pallas_call(
        functools.partial(_ffn_kernel, bm=bm),
        grid_spec=grid_spec,
        out_shape=jax.ShapeDtypeStruct((e, s, d), jnp.float32),
        compiler_params=pltpu.CompilerParams(
            dimension_semantics=("arbitrary", "arbitrary")),
    )(counts, xg3, w1, b1.reshape(e, 1, dff), w2, b2.reshape(e, 1, d))


def _combine_ln_kernel(x_ref, y1_ref, y2_ref, g_ref, gam_ref, bet_ref, o_ref):
    g = g_ref[...]
    iota = lax.broadcasted_iota(jnp.int32, g.shape, 1)
    sel = g > 0.0
    e1 = jnp.min(jnp.where(sel, iota, E), axis=-1, keepdims=True)
    e1 = jnp.minimum(e1, E - 1)
    e2 = jnp.max(jnp.where(sel, iota, 0), axis=-1, keepdims=True)
    w1 = jnp.sum(jnp.where(iota == e1, g, 0.0), axis=-1, keepdims=True)
    w2 = jnp.sum(jnp.where(iota == e2, g, 0.0), axis=-1, keepdims=True)
    w2 = jnp.where(e2 == e1, 0.0, w2)
    x = x_ref[...] + w1 * y1_ref[...] + w2 * y2_ref[...]
    mu = jnp.mean(x, axis=-1, keepdims=True)
    xc = x - mu
    var = jnp.mean(xc * xc, axis=-1, keepdims=True)
    o_ref[...] = xc * lax.rsqrt(var + 1e-5) * gam_ref[...] + bet_ref[...]


def _combine_ln(x, y1, y2, gates, g, b, bm=256):
    m, n = x.shape
    e = gates.shape[1]
    return pl.pallas_call(
        _combine_ln_kernel,
        grid=(m // bm,),
        in_specs=[
            pl.BlockSpec((bm, n), lambda i: (i, 0)),
            pl.BlockSpec((bm, n), lambda i: (i, 0)),
            pl.BlockSpec((bm, n), lambda i: (i, 0)),
            pl.BlockSpec((bm, e), lambda i: (i, 0)),
            pl.BlockSpec((1, n), lambda i: (0, 0)),
            pl.BlockSpec((1, n), lambda i: (0, 0)),
        ],
        out_specs=pl.BlockSpec((bm, n), lambda i: (i, 0)),
        out_shape=jax.ShapeDtypeStruct((m, n), jnp.float32),
    )(x, y1, y2, gates, g.reshape(1, n), b.reshape(1, n))


def _mha(xq, xkv, p, causal=False):
    wqkv = jnp.concatenate([p['wq'], p['wk'], p['wv']], axis=1)
    bqkv = jnp.concatenate([p['bq'], p['bk'], p['bv']], axis=0)
    if xq is xkv:
        qkv = _mm_bias(xq, wqkv, bqkv, bn=768)
        q, k, v = qkv[:, :D], qkv[:, D:2 * D], qkv[:, 2 * D:]
    else:
        q = _mm_bias(xq, p['wq'], p['bq'])
        wkv = jnp.concatenate([p['wk'], p['wv']], axis=1)
        bkv = jnp.concatenate([p['bk'], p['bv']], axis=0)
        kv = _mm_bias(xkv, wkv, bkv, bn=512)
        k, v = kv[:, :D], kv[:, D:]
    ctx = _attention(q, k, v, causal)
    return _mm_bias(ctx, p['wo'], p['bo'])


def _moe_ln_block(x, p, lnp):
    gates, counts = _router(x, p['wr'], p['br'])
    slotpack = _route_plan(gates)
    slot1 = slotpack[:, 0]
    slot2 = slotpack[:, 1]
    xg = _sc_dispatch(x, slot1, slot2)
    cnt_i = counts.reshape(E).astype(jnp.int32)
    yg = _ffn_sparse(xg.reshape(E, S, D), cnt_i,
                     p['w1'], p['b1'], p['w2'], p['b2'])
    y1, y2 = _sc_combine_gather(yg.reshape(E * S, D), slot1, slot2)
    return _combine_ln(x, y1, y2, gates, lnp['g'], lnp['b'])


def kernel(src_ids, tgt_ids, params):
    src = src_ids.reshape(-1).astype(jnp.int32)
    tgt = tgt_ids.reshape(-1).astype(jnp.int32)
    ids = jnp.concatenate([src, tgt], axis=0)
    rows = _sc_embed_gather(params['emb'], ids, 2 * S)
    pe = jnp.asarray(_pe_np(S, D))
    x = _add(rows[:S], pe)
    y0 = _add(rows[S:], pe)

    lp = params['enc'][0]
    a = _mha(x, x, lp['attn'])
    x = _ln_res(x, a, lp['ln1']['g'], lp['ln1']['b'])
    x = _moe_ln_block(x, lp['moe'], lp['ln2'])
    enc_out = x

    lp = params['dec'][0]
    y = y0
    a = _mha(y, y, lp['sattn'], causal=True)
    y = _ln_res(y, a, lp['ln1']['g'], lp['ln1']['b'])
    c = _mha(y, enc_out, lp['cattn'])
    y = _ln_res(y, c, lp['ln2']['g'], lp['ln2']['b'])
    y = _moe_ln_block(y, lp['moe'], lp['ln3'])

    logits = _mm_bias(y, params['wout'], params['bout'], bn=1024)
    return logits.reshape(1, S, V)


# speedup vs baseline: 1.0306x; 1.0306x over previous
import functools
import math

import jax
import jax.numpy as jnp
import numpy as np
from jax import lax
from jax.experimental import pallas as pl
from jax.experimental.pallas import tpu as pltpu
from jax.experimental.pallas import tpu_sc as plsc

V = 16384
D = 768
H = 12
DFF = 1024
E = 8
TOPK = 2
S = 2048
DK = D // H

_NEG = -1e9


def _pe_np(seq_len, d):
    position = np.arange(seq_len, dtype=np.float32)[:, None]
    div_term = np.exp(np.arange(0, d, 2, dtype=np.float32) * (-math.log(10000.0) / d))
    pe = np.zeros((seq_len, d), dtype=np.float32)
    pe[:, 0::2] = np.sin(position * div_term)
    pe[:, 1::2] = np.cos(position * div_term)
    return pe


def _sc_embed_gather(table, ids, n_rows):
    nw = 32
    per_w = n_rows // nw
    mesh = plsc.VectorSubcoreMesh(core_axis_name="c", subcore_axis_name="s")

    @functools.partial(
        pl.kernel,
        mesh=mesh,
        out_type=jax.ShapeDtypeStruct((n_rows, D), jnp.float32),
        scratch_types=[
            pltpu.VMEM((per_w,), jnp.int32),
            pltpu.VMEM((per_w, D), jnp.float32),
            pltpu.SemaphoreType.DMA,
        ],
    )
    def k(table_hbm, ids_hbm, out_hbm, idx_v, rows_v, sem):
        wid = lax.axis_index("s") * 2 + lax.axis_index("c")
        base = wid * per_w
        pltpu.sync_copy(ids_hbm.at[pl.ds(base, per_w)], idx_v)
        pltpu.async_copy(table_hbm.at[idx_v], rows_v, sem).wait()
        pltpu.sync_copy(rows_v, out_hbm.at[pl.ds(base, per_w)])

    return k(table, ids)


def _mm_kernel(x_ref, w_ref, b_ref, o_ref):
    acc = lax.dot_general(
        x_ref[...], w_ref[...], (((1,), (0,)), ((), ())),
        preferred_element_type=jnp.float32)
    o_ref[...] = acc + b_ref[...]


def _mm_bias(x, w, b, bn=256):
    m, k = x.shape
    _, n = w.shape
    return pl.pallas_call(
        _mm_kernel,
        grid=(n // bn,),
        in_specs=[
            pl.BlockSpec((m, k), lambda j: (0, 0)),
            pl.BlockSpec((k, bn), lambda j: (0, j)),
            pl.BlockSpec((1, bn), lambda j: (0, j)),
        ],
        out_specs=pl.BlockSpec((m, bn), lambda j: (0, j)),
        out_shape=jax.ShapeDtypeStruct((m, n), jnp.float32),
        compiler_params=pltpu.CompilerParams(
            dimension_semantics=("arbitrary",)),
    )(x, w, b.reshape(1, n))


def _add_kernel(a_ref, b_ref, o_ref):
    o_ref[...] = a_ref[...] + b_ref[...]


def _add(a, b, bm=256):
    m, n = a.shape
    return pl.pallas_call(
        _add_kernel,
        grid=(m // bm,),
        in_specs=[
            pl.BlockSpec((bm, n), lambda i: (i, 0)),
            pl.BlockSpec((bm, n), lambda i: (i, 0)),
        ],
        out_specs=pl.BlockSpec((bm, n), lambda i: (i, 0)),
        out_shape=jax.ShapeDtypeStruct((m, n), jnp.float32),
    )(a, b)


def _ln_kernel(x_ref, r_ref, g_ref, b_ref, o_ref):
    x = x_ref[...] + r_ref[...]
    mu = jnp.mean(x, axis=-1, keepdims=True)
    xc = x - mu
    var = jnp.mean(xc * xc, axis=-1, keepdims=True)
    o_ref[...] = xc * lax.rsqrt(var + 1e-5) * g_ref[...] + b_ref[...]


def _ln_res(x, r, g, b, bm=256):
    m, n = x.shape
    return pl.pallas_call(
        _ln_kernel,
        grid=(m // bm,),
        in_specs=[
            pl.BlockSpec((bm, n), lambda i: (i, 0)),
            pl.BlockSpec((bm, n), lambda i: (i, 0)),
            pl.BlockSpec((1, n), lambda i: (0, 0)),
            pl.BlockSpec((1, n), lambda i: (0, 0)),
        ],
        out_specs=pl.BlockSpec((bm, n), lambda i: (i, 0)),
        out_shape=jax.ShapeDtypeStruct((m, n), jnp.float32),
    )(x, r, g.reshape(1, n), b.reshape(1, n))


def _attn_kernel(q_ref, k_ref, v_ref, o_ref, *, causal, bq):
    i = pl.program_id(0)
    skv = k_ref.shape[0]
    scale = 1.0 / math.sqrt(DK)
    if not causal:
        for h in range(H):
            sl = pl.ds(h * DK, DK)
            s = lax.dot_general(q_ref[:, sl], k_ref[:, sl],
                                (((1,), (1,)), ((), ())),
                                preferred_element_type=jnp.float32) * scale
            m = jnp.max(s, axis=-1, keepdims=True)
            p = jnp.exp(s - m)
            p = p * (1.0 / jnp.sum(p, axis=-1, keepdims=True))
            o_ref[:, sl] = lax.dot_general(p, v_ref[:, sl],
                                           (((1,), (0,)), ((), ())),
                                           preferred_element_type=jnp.float32)
        return
    rows = i * bq + lax.broadcasted_iota(jnp.int32, (bq, skv), 0)
    cols = lax.broadcasted_iota(jnp.int32, (bq, skv), 1)
    mask = rows >= cols
    for h in range(H):
        sl = pl.ds(h * DK, DK)
        s = lax.dot_general(q_ref[:, sl], k_ref[:, sl],
                            (((1,), (1,)), ((), ())),
                            preferred_element_type=jnp.float32) * scale
        s = jnp.where(mask, s, _NEG)
        m = jnp.max(s, axis=-1, keepdims=True)
        p = jnp.exp(s - m)
        p = p * (1.0 / jnp.sum(p, axis=-1, keepdims=True))
        o_ref[:, sl] = lax.dot_general(p, v_ref[:, sl],
                                       (((1,), (0,)), ((), ())),
                                       preferred_element_type=jnp.float32)


def _attention(q2, k2, v2, causal, bq=256):
    sq, d = q2.shape
    skv = k2.shape[0]
    return pl.pallas_call(
        functools.partial(_attn_kernel, causal=causal, bq=bq),
        grid=(sq // bq,),
        in_specs=[
            pl.BlockSpec((bq, d), lambda i: (i, 0)),
            pl.BlockSpec((skv, d), lambda i: (0, 0)),
            pl.BlockSpec((skv, d), lambda i: (0, 0)),
        ],
        out_specs=pl.BlockSpec((bq, d), lambda i: (i, 0)),
        out_shape=jax.ShapeDtypeStruct((sq, d), jnp.float32),
        compiler_params=pltpu.CompilerParams(
            dimension_semantics=("arbitrary",)),
    )(q2, k2, v2)


def _router_kernel(x_ref, wr_ref, br_ref, g_ref, c_ref):
    logits = lax.dot_general(x_ref[...], wr_ref[...], (((1,), (0,)), ((), ())),
                             preferred_element_type=jnp.float32) + br_ref[...]
    mx = jnp.max(logits, axis=-1, keepdims=True)
    ex = jnp.exp(logits - mx)
    probs = ex / jnp.sum(ex, axis=-1, keepdims=True)
    iota = lax.broadcasted_iota(jnp.int32, probs.shape, 1)
    m1 = jnp.max(probs, axis=-1, keepdims=True)
    i1 = jnp.min(jnp.where(probs == m1, iota, E), axis=-1, keepdims=True)
    oh1 = iota == i1
    p2 = jnp.where(oh1, -1.0, probs)
    m2 = jnp.max(p2, axis=-1, keepdims=True)
    i2 = jnp.min(jnp.where(p2 == m2, iota, E), axis=-1, keepdims=True)
    oh2 = iota == i2
    denom = m1 + m2
    g_ref[...] = (jnp.where(oh1, m1, 0.0) + jnp.where(oh2, m2, 0.0)) / denom

    blockcnt = jnp.sum(jnp.where(oh1 | oh2, 1.0, 0.0), axis=0, keepdims=True)

    @pl.when(pl.program_id(0) == 0)
    def _():
        c_ref[...] = jnp.zeros_like(c_ref)

    c_ref[...] += blockcnt


def _router(x, wr, br, bm=256):
    m, k = x.shape
    e = wr.shape[1]
    return pl.pallas_call(
        _router_kernel,
        grid=(m // bm,),
        in_specs=[
            pl.BlockSpec((bm, k), lambda i: (i, 0)),
            pl.BlockSpec((k, e), lambda i: (0, 0)),
            pl.BlockSpec((1, e), lambda i: (0, 0)),
        ],
        out_specs=[
            pl.BlockSpec((bm, e), lambda i: (i, 0)),
            pl.BlockSpec((1, e), lambda i: (0, 0)),
        ],
        out_shape=[
            jax.ShapeDtypeStruct((m, e), jnp.float32),
            jax.ShapeDtypeStruct((1, e), jnp.float32),
        ],
        compiler_params=pltpu.CompilerParams(
            dimension_semantics=("arbitrary",)),
    )(x, wr, br.reshape(1, e))


def _plan_kernel(gf_ref, gb_ref, s_ref, *, bm, s_tot):
    i = pl.program_id(0)
    m = jnp.where(gf_ref[...] > 0.0, 1.0, 0.0)
    rows = i * bm + lax.broadcasted_iota(jnp.int32, (bm, s_tot), 0)
    cols = lax.broadcasted_iota(jnp.int32, (bm, s_tot), 1)
    lt = jnp.where(cols < rows, 1.0, 0.0)
    cnt = lax.dot_general(lt, m, (((1,), (0,)), ((), ())),
                          preferred_element_type=jnp.float32)
    g = gb_ref[...]
    iota = lax.broadcasted_iota(jnp.int32, g.shape, 1)
    sel = g > 0.0
    e1 = jnp.min(jnp.where(sel, iota, E), axis=-1, keepdims=True)
    e1 = jnp.minimum(e1, E - 1)
    e2 = jnp.max(jnp.where(sel, iota, 0), axis=-1, keepdims=True)
    rank1 = jnp.sum(jnp.where(iota == e1, cnt, 0.0), axis=-1, keepdims=True)
    rank2 = jnp.sum(jnp.where(iota == e2, cnt, 0.0), axis=-1, keepdims=True)
    slot1 = e1 * s_tot + rank1.astype(jnp.int32)
    slot2 = e2 * s_tot + rank2.astype(jnp.int32)
    s_ref[...] = (jnp.where(iota == 0, slot1, 0)
                  + jnp.where(iota == 1, slot2, 0))


def _route_plan(gates, bm=256):
    m, e = gates.shape
    return pl.pallas_call(
        functools.partial(_plan_kernel, bm=bm, s_tot=m),
        grid=(m // bm,),
        in_specs=[
            pl.BlockSpec((m, e), lambda i: (0, 0)),
            pl.BlockSpec((bm, e), lambda i: (i, 0)),
        ],
        out_specs=pl.BlockSpec((bm, e), lambda i: (i, 0)),
        out_shape=jax.ShapeDtypeStruct((m, e), jnp.int32),
    )(gates, gates)


def _sc_dispatch(x, slot1, slot2):
    nw = 32
    per_w = S // nw
    mesh = plsc.VectorSubcoreMesh(core_axis_name="c", subcore_axis_name="s")

    @functools.partial(
        pl.kernel,
        mesh=mesh,
        out_type=jax.ShapeDtypeStruct((E * S, D), jnp.float32),
        scratch_types=[
            pltpu.VMEM((per_w,), jnp.int32),
            pltpu.VMEM((per_w,), jnp.int32),
            pltpu.VMEM((per_w, D), jnp.float32),
            pltpu.SemaphoreType.DMA,
            pltpu.SemaphoreType.DMA,
        ],
    )
    def k(x_hbm, s1_hbm, s2_hbm, xg_hbm, i1_v, i2_v, rows_v, sem1, sem2):
        wid = lax.axis_index("s") * 2 + lax.axis_index("c")
        base = wid * per_w
        pltpu.sync_copy(x_hbm.at[pl.ds(base, per_w)], rows_v)
        pltpu.sync_copy(s1_hbm.at[pl.ds(base, per_w)], i1_v)
        pltpu.sync_copy(s2_hbm.at[pl.ds(base, per_w)], i2_v)
        c1 = pltpu.async_copy(rows_v, xg_hbm.at[i1_v], sem1)
        c2 = pltpu.async_copy(rows_v, xg_hbm.at[i2_v], sem2)
        c1.wait()
        c2.wait()

    return k(x, slot1, slot2)


def _sc_combine_gather(yg, slot1, slot2):
    nw = 32
    per_w = S // nw
    mesh = plsc.VectorSubcoreMesh(core_axis_name="c", subcore_axis_name="s")

    @functools.partial(
        pl.kernel,
        mesh=mesh,
        out_type=[
            jax.ShapeDtypeStruct((S, D), jnp.float32),
            jax.ShapeDtypeStruct((S, D), jnp.float32),
        ],
        scratch_types=[
            pltpu.VMEM((per_w,), jnp.int32),
            pltpu.VMEM((per_w,), jnp.int32),
            pltpu.VMEM((per_w, D), jnp.float32),
            pltpu.VMEM((per_w, D), jnp.float32),
            pltpu.SemaphoreType.DMA,
            pltpu.SemaphoreType.DMA,
        ],
    )
    def k(yg_hbm, s1_hbm, s2_hbm, y1_hbm, y2_hbm,
          i1_v, i2_v, r1_v, r2_v, sem1, sem2):
        wid = lax.axis_index("s") * 2 + lax.axis_index("c")
        base = wid * per_w
        pltpu.sync_copy(s1_hbm.at[pl.ds(base, per_w)], i1_v)
        pltpu.sync_copy(s2_hbm.at[pl.ds(base, per_w)], i2_v)
        c1 = pltpu.async_copy(yg_hbm.at[i1_v], r1_v, sem1)
        c2 = pltpu.async_copy(yg_hbm.at[i2_v], r2_v, sem2)
        c1.wait()
        c2.wait()
        pltpu.sync_copy(r1_v, y1_hbm.at[pl.ds(base, per_w)])
        pltpu.sync_copy(r2_v, y2_hbm.at[pl.ds(base, per_w)])

    return k(yg, slot1, slot2)


def _ffn_kernel(cnt_ref, xg_ref, w1_ref, b1_ref, w2_ref, b2_ref, o_ref, *, bm):
    e = pl.program_id(0)
    j = pl.program_id(1)

    @pl.when(j * bm < cnt_ref[e])
    def _():
        h1 = lax.dot_general(xg_ref[0], w1_ref[0], (((1,), (0,)), ((), ())),
                             preferred_element_type=jnp.float32) + b1_ref[0]
        h1 = jnp.maximum(h1, 0.0)
        o_ref[0] = lax.dot_general(h1, w2_ref[0], (((1,), (0,)), ((), ())),
                                   preferred_element_type=jnp.float32) + b2_ref[0]


def _ffn_sparse(xg3, counts, w1, b1, w2, b2, bm=256):
    e, s, d = xg3.shape
    dff = w1.shape[2]

    def xg_map(ei, j, cnt):
        nb = jnp.maximum(pl.cdiv(cnt[ei], bm) - 1, 0)
        return (ei, jnp.minimum(j, nb), 0)

    grid_spec = pltpu.PrefetchScalarGridSpec(
        num_scalar_prefetch=1,
        grid=(e, s // bm),
        in_specs=[
            pl.BlockSpec((1, bm, d), xg_map),
            pl.BlockSpec((1, d, dff), lambda ei, j, cnt: (ei, 0, 0)),
            pl.BlockSpec((1, 1, dff), lambda ei, j, cnt: (ei, 0, 0)),
            pl.BlockSpec((1, dff, d), lambda ei, j, cnt: (ei, 0, 0)),
            pl.BlockSpec((1, 1, d), lambda ei, j, cnt: (ei, 0, 0)),
        ],
        out_specs=pl.BlockSpec((1, bm, d), xg_map),
    )
    return pl.pallas_call(
        functools.partial(_ffn_kernel, bm=bm),
        grid_spec=grid_spec,
        out_shape=jax.ShapeDtypeStruct((e, s, d), jnp.float32),
        compiler_params=pltpu.CompilerParams(
            dimension_semantics=("arbitrary", "arbitrary")),
    )(counts, xg3, w1, b1.reshape(e, 1, dff), w2, b2.reshape(e, 1, d))


def _combine_ln_kernel(x_ref, y1_ref, y2_ref, g_ref, gam_ref, bet_ref, o_ref):
    g = g_ref[...]
    iota = lax.broadcasted_iota(jnp.int32, g.shape, 1)
    sel = g > 0.0
    e1 = jnp.min(jnp.where(sel, iota, E), axis=-1, keepdims=True)
    e1 = jnp.minimum(e1, E - 1)
    e2 = jnp.max(jnp.where(sel, iota, 0), axis=-1, keepdims=True)
    w1 = jnp.sum(jnp.where(iota == e1, g, 0.0), axis=-1, keepdims=True)
    w2 = jnp.sum(jnp.where(iota == e2, g, 0.0), axis=-1, keepdims=True)
    w2 = jnp.where(e2 == e1, 0.0, w2)
    x = x_ref[...] + w1 * y1_ref[...] + w2 * y2_ref[...]
    mu = jnp.mean(x, axis=-1, keepdims=True)
    xc = x - mu
    var = jnp.mean(xc * xc, axis=-1, keepdims=True)
    o_ref[...] = xc * lax.rsqrt(var + 1e-5) * gam_ref[...] + bet_ref[...]


def _combine_ln(x, y1, y2, gates, g, b, bm=256):
    m, n = x.shape
    e = gates.shape[1]
    return pl.pallas_call(
        _combine_ln_kernel,
        grid=(m // bm,),
        in_specs=[
            pl.BlockSpec((bm, n), lambda i: (i, 0)),
            pl.BlockSpec((bm, n), lambda i: (i, 0)),
            pl.BlockSpec((bm, n), lambda i: (i, 0)),
            pl.BlockSpec((bm, e), lambda i: (i, 0)),
            pl.BlockSpec((1, n), lambda i: (0, 0)),
            pl.BlockSpec((1, n), lambda i: (0, 0)),
        ],
        out_specs=pl.BlockSpec((bm, n), lambda i: (i, 0)),
        out_shape=jax.ShapeDtypeStruct((m, n), jnp.float32),
    )(x, y1, y2, gates, g.reshape(1, n), b.reshape(1, n))


def _mha(xq, xkv, p, causal=False):
    wqkv = jnp.concatenate([p['wq'], p['wk'], p['wv']], axis=1)
    bqkv = jnp.concatenate([p['bq'], p['bk'], p['bv']], axis=0)
    if xq is xkv:
        qkv = _mm_bias(xq, wqkv, bqkv, bn=768)
        q, k, v = qkv[:, :D], qkv[:, D:2 * D], qkv[:, 2 * D:]
    else:
        q = _mm_bias(xq, p['wq'], p['bq'])
        wkv = jnp.concatenate([p['wk'], p['wv']], axis=1)
        bkv = jnp.concatenate([p['bk'], p['bv']], axis=0)
        kv = _mm_bias(xkv, wkv, bkv, bn=512)
        k, v = kv[:, :D], kv[:, D:]
    ctx = _attention(q, k, v, causal)
    return _mm_bias(ctx, p['wo'], p['bo'])


def _moe_ln_block(x, p, lnp):
    gates, counts = _router(x, p['wr'], p['br'])
    slotpack = _route_plan(gates)
    slot1 = slotpack[:, 0]
    slot2 = slotpack[:, 1]
    xg = _sc_dispatch(x, slot1, slot2)
    cnt_i = counts.reshape(E).astype(jnp.int32)
    yg = _ffn_sparse(xg.reshape(E, S, D), cnt_i,
                     p['w1'], p['b1'], p['w2'], p['b2'])
    y1, y2 = _sc_combine_gather(yg.reshape(E * S, D), slot1, slot2)
    return _combine_ln(x, y1, y2, gates, lnp['g'], lnp['b'])


def kernel(src_ids, tgt_ids, params):
    src = src_ids.reshape(-1).astype(jnp.int32)
    tgt = tgt_ids.reshape(-1).astype(jnp.int32)
    ids = jnp.concatenate([src, tgt], axis=0)
    rows = _sc_embed_gather(params['emb'], ids, 2 * S)
    pe = jnp.asarray(_pe_np(S, D))
    x = _add(rows[:S], pe)
    y0 = _add(rows[S:], pe)

    lp = params['enc'][0]
    a = _mha(x, x, lp['attn'])
    x = _ln_res(x, a, lp['ln1']['g'], lp['ln1']['b'])
    x = _moe_ln_block(x, lp['moe'], lp['ln2'])
    enc_out = x

    lp = params['dec'][0]
    y = y0
    a = _mha(y, y, lp['sattn'], causal=True)
    y = _ln_res(y, a, lp['ln1']['g'], lp['ln1']['b'])
    c = _mha(y, enc_out, lp['cattn'])
    y = _ln_res(y, c, lp['ln2']['g'], lp['ln2']['b'])
    y = _moe_ln_block(y, lp['moe'], lp['ln3'])

    logits = _mm_bias(y, params['wout'], params['bout'], bn=1024)
    return logits.reshape(1, S, V)


# speedup vs baseline: 1.0308x; 1.0002x over previous
import functools
import math

import jax
import jax.numpy as jnp
import numpy as np
from jax import lax
from jax.experimental import pallas as pl
from jax.experimental.pallas import tpu as pltpu
from jax.experimental.pallas import tpu_sc as plsc

V = 16384
D = 768
H = 12
DFF = 1024
E = 8
TOPK = 2
S = 2048
DK = D // H

_NEG = -1e9


def _pe_np(seq_len, d):
    position = np.arange(seq_len, dtype=np.float32)[:, None]
    div_term = np.exp(np.arange(0, d, 2, dtype=np.float32) * (-math.log(10000.0) / d))
    pe = np.zeros((seq_len, d), dtype=np.float32)
    pe[:, 0::2] = np.sin(position * div_term)
    pe[:, 1::2] = np.cos(position * div_term)
    return pe


def _sc_embed_gather(table, ids, n_rows):
    nw = 32
    per_w = n_rows // nw
    mesh = plsc.VectorSubcoreMesh(core_axis_name="c", subcore_axis_name="s")

    @functools.partial(
        pl.kernel,
        mesh=mesh,
        out_type=jax.ShapeDtypeStruct((n_rows, D), jnp.float32),
        scratch_types=[
            pltpu.VMEM((per_w,), jnp.int32),
            pltpu.VMEM((per_w, D), jnp.float32),
            pltpu.SemaphoreType.DMA,
        ],
    )
    def k(table_hbm, ids_hbm, out_hbm, idx_v, rows_v, sem):
        wid = lax.axis_index("s") * 2 + lax.axis_index("c")
        base = wid * per_w
        pltpu.sync_copy(ids_hbm.at[pl.ds(base, per_w)], idx_v)
        pltpu.async_copy(table_hbm.at[idx_v], rows_v, sem).wait()
        pltpu.sync_copy(rows_v, out_hbm.at[pl.ds(base, per_w)])

    return k(table, ids)


def _mm_kernel(x_ref, w_ref, b_ref, o_ref):
    acc = lax.dot_general(
        x_ref[...], w_ref[...], (((1,), (0,)), ((), ())),
        preferred_element_type=jnp.float32)
    o_ref[...] = acc + b_ref[...]


def _mm_bias(x, w, b, bn=256):
    m, k = x.shape
    _, n = w.shape
    return pl.pallas_call(
        _mm_kernel,
        grid=(n // bn,),
        in_specs=[
            pl.BlockSpec((m, k), lambda j: (0, 0)),
            pl.BlockSpec((k, bn), lambda j: (0, j)),
            pl.BlockSpec((1, bn), lambda j: (0, j)),
        ],
        out_specs=pl.BlockSpec((m, bn), lambda j: (0, j)),
        out_shape=jax.ShapeDtypeStruct((m, n), jnp.float32),
        compiler_params=pltpu.CompilerParams(
            dimension_semantics=("arbitrary",)),
    )(x, w, b.reshape(1, n))


def _add_kernel(a_ref, b_ref, o_ref):
    o_ref[...] = a_ref[...] + b_ref[...]


def _add(a, b, bm=256):
    m, n = a.shape
    return pl.pallas_call(
        _add_kernel,
        grid=(m // bm,),
        in_specs=[
            pl.BlockSpec((bm, n), lambda i: (i, 0)),
            pl.BlockSpec((bm, n), lambda i: (i, 0)),
        ],
        out_specs=pl.BlockSpec((bm, n), lambda i: (i, 0)),
        out_shape=jax.ShapeDtypeStruct((m, n), jnp.float32),
    )(a, b)


def _ln_kernel(x_ref, r_ref, g_ref, b_ref, o_ref):
    x = x_ref[...] + r_ref[...]
    mu = jnp.mean(x, axis=-1, keepdims=True)
    xc = x - mu
    var = jnp.mean(xc * xc, axis=-1, keepdims=True)
    o_ref[...] = xc * lax.rsqrt(var + 1e-5) * g_ref[...] + b_ref[...]


def _ln_res(x, r, g, b, bm=256):
    m, n = x.shape
    return pl.pallas_call(
        _ln_kernel,
        grid=(m // bm,),
        in_specs=[
            pl.BlockSpec((bm, n), lambda i: (i, 0)),
            pl.BlockSpec((bm, n), lambda i: (i, 0)),
            pl.BlockSpec((1, n), lambda i: (0, 0)),
            pl.BlockSpec((1, n), lambda i: (0, 0)),
        ],
        out_specs=pl.BlockSpec((bm, n), lambda i: (i, 0)),
        out_shape=jax.ShapeDtypeStruct((m, n), jnp.float32),
    )(x, r, g.reshape(1, n), b.reshape(1, n))


def _attn_kernel(q_ref, k_ref, v_ref, o_ref, *, causal, bq):
    i = pl.program_id(0)
    skv = k_ref.shape[0]
    scale = 1.0 / math.sqrt(DK)
    if not causal:
        for h in range(H):
            sl = pl.ds(h * DK, DK)
            s = lax.dot_general(q_ref[:, sl], k_ref[:, sl],
                                (((1,), (1,)), ((), ())),
                                preferred_element_type=jnp.float32) * scale
            m = jnp.max(s, axis=-1, keepdims=True)
            p = jnp.exp(s - m)
            p = p * (1.0 / jnp.sum(p, axis=-1, keepdims=True))
            o_ref[:, sl] = lax.dot_general(p, v_ref[:, sl],
                                           (((1,), (0,)), ((), ())),
                                           preferred_element_type=jnp.float32)
        return
    rows = i * bq + lax.broadcasted_iota(jnp.int32, (bq, skv), 0)
    cols = lax.broadcasted_iota(jnp.int32, (bq, skv), 1)
    mask = rows >= cols
    for h in range(H):
        sl = pl.ds(h * DK, DK)
        s = lax.dot_general(q_ref[:, sl], k_ref[:, sl],
                            (((1,), (1,)), ((), ())),
                            preferred_element_type=jnp.float32) * scale
        s = jnp.where(mask, s, _NEG)
        m = jnp.max(s, axis=-1, keepdims=True)
        p = jnp.exp(s - m)
        p = p * (1.0 / jnp.sum(p, axis=-1, keepdims=True))
        o_ref[:, sl] = lax.dot_general(p, v_ref[:, sl],
                                       (((1,), (0,)), ((), ())),
                                       preferred_element_type=jnp.float32)


def _attention(q2, k2, v2, causal, bq=256):
    sq, d = q2.shape
    skv = k2.shape[0]
    return pl.pallas_call(
        functools.partial(_attn_kernel, causal=causal, bq=bq),
        grid=(sq // bq,),
        in_specs=[
            pl.BlockSpec((bq, d), lambda i: (i, 0)),
            pl.BlockSpec((skv, d), lambda i: (0, 0)),
            pl.BlockSpec((skv, d), lambda i: (0, 0)),
        ],
        out_specs=pl.BlockSpec((bq, d), lambda i: (i, 0)),
        out_shape=jax.ShapeDtypeStruct((sq, d), jnp.float32),
        compiler_params=pltpu.CompilerParams(
            dimension_semantics=("arbitrary",)),
    )(q2, k2, v2)


def _router_kernel(x_ref, wr_ref, br_ref, g_ref, c_ref):
    logits = lax.dot_general(x_ref[...], wr_ref[...], (((1,), (0,)), ((), ())),
                             preferred_element_type=jnp.float32) + br_ref[...]
    mx = jnp.max(logits, axis=-1, keepdims=True)
    ex = jnp.exp(logits - mx)
    probs = ex / jnp.sum(ex, axis=-1, keepdims=True)
    iota = lax.broadcasted_iota(jnp.int32, probs.shape, 1)
    m1 = jnp.max(probs, axis=-1, keepdims=True)
    i1 = jnp.min(jnp.where(probs == m1, iota, E), axis=-1, keepdims=True)
    oh1 = iota == i1
    p2 = jnp.where(oh1, -1.0, probs)
    m2 = jnp.max(p2, axis=-1, keepdims=True)
    i2 = jnp.min(jnp.where(p2 == m2, iota, E), axis=-1, keepdims=True)
    oh2 = iota == i2
    denom = m1 + m2
    g_ref[...] = (jnp.where(oh1, m1, 0.0) + jnp.where(oh2, m2, 0.0)) / denom

    blockcnt = jnp.sum(jnp.where(oh1 | oh2, 1.0, 0.0), axis=0, keepdims=True)

    @pl.when(pl.program_id(0) == 0)
    def _():
        c_ref[...] = jnp.zeros_like(c_ref)

    c_ref[...] += blockcnt


def _router(x, wr, br, bm=256):
    m, k = x.shape
    e = wr.shape[1]
    return pl.pallas_call(
        _router_kernel,
        grid=(m // bm,),
        in_specs=[
            pl.BlockSpec((bm, k), lambda i: (i, 0)),
            pl.BlockSpec((k, e), lambda i: (0, 0)),
            pl.BlockSpec((1, e), lambda i: (0, 0)),
        ],
        out_specs=[
            pl.BlockSpec((bm, e), lambda i: (i, 0)),
            pl.BlockSpec((1, e), lambda i: (0, 0)),
        ],
        out_shape=[
            jax.ShapeDtypeStruct((m, e), jnp.float32),
            jax.ShapeDtypeStruct((1, e), jnp.float32),
        ],
        compiler_params=pltpu.CompilerParams(
            dimension_semantics=("arbitrary",)),
    )(x, wr, br.reshape(1, e))


def _plan_kernel(gf_ref, gb_ref, s_ref, *, bm, s_tot):
    i = pl.program_id(0)
    m = jnp.where(gf_ref[...] > 0.0, 1.0, 0.0)
    rows = i * bm + lax.broadcasted_iota(jnp.int32, (bm, s_tot), 0)
    cols = lax.broadcasted_iota(jnp.int32, (bm, s_tot), 1)
    lt = jnp.where(cols < rows, 1.0, 0.0)
    cnt = lax.dot_general(lt, m, (((1,), (0,)), ((), ())),
                          preferred_element_type=jnp.float32)
    g = gb_ref[...]
    iota = lax.broadcasted_iota(jnp.int32, g.shape, 1)
    sel = g > 0.0
    e1 = jnp.min(jnp.where(sel, iota, E), axis=-1, keepdims=True)
    e1 = jnp.minimum(e1, E - 1)
    e2 = jnp.max(jnp.where(sel, iota, 0), axis=-1, keepdims=True)
    rank1 = jnp.sum(jnp.where(iota == e1, cnt, 0.0), axis=-1, keepdims=True)
    rank2 = jnp.sum(jnp.where(iota == e2, cnt, 0.0), axis=-1, keepdims=True)
    slot1 = e1 * s_tot + rank1.astype(jnp.int32)
    slot2 = e2 * s_tot + rank2.astype(jnp.int32)
    s_ref[...] = (jnp.where(iota == 0, slot1, 0)
                  + jnp.where(iota == 1, slot2, 0))


def _route_plan(gates, bm=256):
    m, e = gates.shape
    return pl.pallas_call(
        functools.partial(_plan_kernel, bm=bm, s_tot=m),
        grid=(m // bm,),
        in_specs=[
            pl.BlockSpec((m, e), lambda i: (0, 0)),
            pl.BlockSpec((bm, e), lambda i: (i, 0)),
        ],
        out_specs=pl.BlockSpec((bm, e), lambda i: (i, 0)),
        out_shape=jax.ShapeDtypeStruct((m, e), jnp.int32),
    )(gates, gates)


def _sc_dispatch(x, i1, i2):
    nw = 32
    per_w = S // nw
    mesh = plsc.VectorSubcoreMesh(core_axis_name="c", subcore_axis_name="s")

    @functools.partial(
        pl.kernel,
        mesh=mesh,
        out_type=jax.ShapeDtypeStruct((E * S, D), jnp.float32),
        scratch_types=[
            pltpu.VMEM((per_w,), jnp.int32),
            pltpu.VMEM((per_w,), jnp.int32),
            pltpu.VMEM((per_w, D), jnp.float32),
            pltpu.SemaphoreType.DMA,
            pltpu.SemaphoreType.DMA,
        ],
    )
    def k(x_hbm, i1_hbm, i2_hbm, xg_hbm, i1_v, i2_v, rows_v, sem1, sem2):
        wid = lax.axis_index("s") * 2 + lax.axis_index("c")
        base = wid * per_w
        pltpu.sync_copy(x_hbm.at[pl.ds(base, per_w)], rows_v)
        pltpu.sync_copy(i1_hbm.at[pl.ds(base, per_w)], i1_v)
        pltpu.sync_copy(i2_hbm.at[pl.ds(base, per_w)], i2_v)
        c1 = pltpu.async_copy(rows_v, xg_hbm.at[i1_v], sem1)
        c2 = pltpu.async_copy(rows_v, xg_hbm.at[i2_v], sem2)
        c1.wait()
        c2.wait()

    return k(x, i1, i2)


def _sc_combine_gather(yg, i1, i2):
    nw = 32
    per_w = S // nw
    mesh = plsc.VectorSubcoreMesh(core_axis_name="c", subcore_axis_name="s")

    @functools.partial(
        pl.kernel,
        mesh=mesh,
        out_type=[
            jax.ShapeDtypeStruct((S, D), jnp.float32),
            jax.ShapeDtypeStruct((S, D), jnp.float32),
        ],
        scratch_types=[
            pltpu.VMEM((per_w,), jnp.int32),
            pltpu.VMEM((per_w,), jnp.int32),
            pltpu.VMEM((per_w, D), jnp.float32),
            pltpu.VMEM((per_w, D), jnp.float32),
            pltpu.SemaphoreType.DMA,
            pltpu.SemaphoreType.DMA,
        ],
    )
    def k(yg_hbm, i1_hbm, i2_hbm, y1_hbm, y2_hbm,
          i1_v, i2_v, r1_v, r2_v, sem1, sem2):
        wid = lax.axis_index("s") * 2 + lax.axis_index("c")
        base = wid * per_w
        pltpu.sync_copy(i1_hbm.at[pl.ds(base, per_w)], i1_v)
        pltpu.sync_copy(i2_hbm.at[pl.ds(base, per_w)], i2_v)
        c1 = pltpu.async_copy(yg_hbm.at[i1_v], r1_v, sem1)
        c2 = pltpu.async_copy(yg_hbm.at[i2_v], r2_v, sem2)
        c1.wait()
        c2.wait()
        pltpu.sync_copy(r1_v, y1_hbm.at[pl.ds(base, per_w)])
        pltpu.sync_copy(r2_v, y2_hbm.at[pl.ds(base, per_w)])

    return k(yg, i1, i2)


def _ffn_kernel(cnt_ref, xg_ref, w1_ref, b1_ref, w2_ref, b2_ref, o_ref, *, bm):
    e = pl.program_id(0)
    j = pl.program_id(1)

    @pl.when(j * bm < cnt_ref[e])
    def _():
        h1 = lax.dot_general(xg_ref[0], w1_ref[0], (((1,), (0,)), ((), ())),
                             preferred_element_type=jnp.float32) + b1_ref[0]
        h1 = jnp.maximum(h1, 0.0)
        o_ref[0] = lax.dot_general(h1, w2_ref[0], (((1,), (0,)), ((), ())),
                                   preferred_element_type=jnp.float32) + b2_ref[0]


def _ffn_sparse(xg3, counts, w1, b1, w2, b2, bm=256):
    e, s, d = xg3.shape
    dff = w1.shape[2]

    def xg_map(ei, j, cnt):
        nb = jnp.maximum(pl.cdiv(cnt[ei], bm) - 1, 0)
        return (ei, jnp.minimum(j, nb), 0)

    grid_spec = pltpu.PrefetchScalarGridSpec(
        num_scalar_prefetch=1,
        grid=(e, s // bm),
        in_specs=[
            pl.BlockSpec((1, bm, d), xg_map),
            pl.BlockSpec((1, d, dff), lambda ei, j, cnt: (ei, 0, 0)),
            pl.BlockSpec((1, 1, dff), lambda ei, j, cnt: (ei, 0, 0)),
            pl.BlockSpec((1, dff, d), lambda ei, j, cnt: (ei, 0, 0)),
            pl.BlockSpec((1, 1, d), lambda ei, j, cnt: (ei, 0, 0)),
        ],
        out_specs=pl.BlockSpec((1, bm, d), xg_map),
    )
    return pl.pallas_call(
        functools.partial(_ffn_kernel, bm=bm),
        grid_spec=grid_spec,
        out_shape=jax.ShapeDtypeStruct((e, s, d), jnp.float32),
        compiler_params=pltpu.CompilerParams(
            dimension_semantics=("arbitrary", "arbitrary")),
    )(counts, xg3, w1, b1.reshape(e, 1, dff), w2, b2.reshape(e, 1, d))


def _combine_ln_kernel(x_ref, y1_ref, y2_ref, g_ref, gam_ref, bet_ref, o_ref):
    g = g_ref[...]
    iota = lax.broadcasted_iota(jnp.int32, g.shape, 1)
    sel = g > 0.0
    e1 = jnp.min(jnp.where(sel, iota, E), axis=-1, keepdims=True)
    e1 = jnp.minimum(e1, E - 1)
    e2 = jnp.max(jnp.where(sel, iota, 0), axis=-1, keepdims=True)
    w1 = jnp.sum(jnp.where(iota == e1, g, 0.0), axis=-1, keepdims=True)
    w2 = jnp.sum(jnp.where(iota == e2, g, 0.0), axis=-1, keepdims=True)
    w2 = jnp.where(e2 == e1, 0.0, w2)
    x = x_ref[...] + w1 * y1_ref[...] + w2 * y2_ref[...]
    mu = jnp.mean(x, axis=-1, keepdims=True)
    xc = x - mu
    var = jnp.mean(xc * xc, axis=-1, keepdims=True)
    o_ref[...] = xc * lax.rsqrt(var + 1e-5) * gam_ref[...] + bet_ref[...]


def _combine_ln(x, y1, y2, gates, g, b, bm=256):
    m, n = x.shape
    e = gates.shape[1]
    return pl.pallas_call(
        _combine_ln_kernel,
        grid=(m // bm,),
        in_specs=[
            pl.BlockSpec((bm, n), lambda i: (i, 0)),
            pl.BlockSpec((bm, n), lambda i: (i, 0)),
            pl.BlockSpec((bm, n), lambda i: (i, 0)),
            pl.BlockSpec((bm, e), lambda i: (i, 0)),
            pl.BlockSpec((1, n), lambda i: (0, 0)),
            pl.BlockSpec((1, n), lambda i: (0, 0)),
        ],
        out_specs=pl.BlockSpec((bm, n), lambda i: (i, 0)),
        out_shape=jax.ShapeDtypeStruct((m, n), jnp.float32),
    )(x, y1, y2, gates, g.reshape(1, n), b.reshape(1, n))


def _mha(xq, xkv, p, causal=False):
    wqkv = jnp.concatenate([p['wq'], p['wk'], p['wv']], axis=1)
    bqkv = jnp.concatenate([p['bq'], p['bk'], p['bv']], axis=0)
    if xq is xkv:
        qkv = _mm_bias(xq, wqkv, bqkv, bn=768)
        q, k, v = qkv[:, :D], qkv[:, D:2 * D], qkv[:, 2 * D:]
    else:
        q = _mm_bias(xq, p['wq'], p['bq'])
        wkv = jnp.concatenate([p['wk'], p['wv']], axis=1)
        bkv = jnp.concatenate([p['bk'], p['bv']], axis=0)
        kv = _mm_bias(xkv, wkv, bkv, bn=512)
        k, v = kv[:, :D], kv[:, D:]
    ctx = _attention(q, k, v, causal)
    return _mm_bias(ctx, p['wo'], p['bo'])


def _moe_ln_block(x, p, lnp):
    gates, counts = _router(x, p['wr'], p['br'])
    slotpack = _route_plan(gates)
    i1 = slotpack[:, 0]
    i2 = slotpack[:, 1]
    xg = _sc_dispatch(x, i1, i2)
    cnt_i = counts.reshape(E).astype(jnp.int32)
    yg = _ffn_sparse(xg.reshape(E, S, D), cnt_i,
                     p['w1'], p['b1'], p['w2'], p['b2'])
    y1, y2 = _sc_combine_gather(yg.reshape(E * S, D), i1, i2)
    return _combine_ln(x, y1, y2, gates, lnp['g'], lnp['b'])


def kernel(src_ids, tgt_ids, params):
    src = src_ids.reshape(-1).astype(jnp.int32)
    tgt = tgt_ids.reshape(-1).astype(jnp.int32)
    ids = jnp.concatenate([src, tgt], axis=0)
    rows = _sc_embed_gather(params['emb'], ids, 2 * S)
    pe = jnp.asarray(_pe_np(S, D))
    x = _add(rows[:S], pe)
    y0 = _add(rows[S:], pe)

    lp = params['enc'][0]
    a = _mha(x, x, lp['attn'])
    x = _ln_res(x, a, lp['ln1']['g'], lp['ln1']['b'])
    x = _moe_ln_block(x, lp['moe'], lp['ln2'])
    enc_out = x

    lp = params['dec'][0]
    y = y0
    a = _mha(y, y, lp['sattn'], causal=True)
    y = _ln_res(y, a, lp['ln1']['g'], lp['ln1']['b'])
    c = _mha(y, enc_out, lp['cattn'])
    y = _ln_res(y, c, lp['ln2']['g'], lp['ln2']['b'])
    y = _moe_ln_block(y, lp['moe'], lp['ln3'])

    logits = _mm_bias(y, params['wout'], params['bout'], bn=1024)
    return logits.reshape(1, S, V)


# speedup vs baseline: 1.0422x; 1.0110x over previous
import functools
import math

import jax
import jax.numpy as jnp
import numpy as np
from jax import lax
from jax.experimental import pallas as pl
from jax.experimental.pallas import tpu as pltpu
from jax.experimental.pallas import tpu_sc as plsc

V = 16384
D = 768
H = 12
DFF = 1024
E = 8
TOPK = 2
S = 2048
DK = D // H

_NEG = -1e9


def _pe_np(seq_len, d):
    position = np.arange(seq_len, dtype=np.float32)[:, None]
    div_term = np.exp(np.arange(0, d, 2, dtype=np.float32) * (-math.log(10000.0) / d))
    pe = np.zeros((seq_len, d), dtype=np.float32)
    pe[:, 0::2] = np.sin(position * div_term)
    pe[:, 1::2] = np.cos(position * div_term)
    return pe


def _sc_embed_gather(table, ids, n_rows):
    nw = 32
    per_w = n_rows // nw
    mesh = plsc.VectorSubcoreMesh(core_axis_name="c", subcore_axis_name="s")

    @functools.partial(
        pl.kernel,
        mesh=mesh,
        out_type=jax.ShapeDtypeStruct((n_rows, D), jnp.float32),
        scratch_types=[
            pltpu.VMEM((per_w,), jnp.int32),
            pltpu.VMEM((per_w, D), jnp.float32),
            pltpu.SemaphoreType.DMA,
        ],
    )
    def k(table_hbm, ids_hbm, out_hbm, idx_v, rows_v, sem):
        wid = lax.axis_index("s") * 2 + lax.axis_index("c")
        base = wid * per_w
        pltpu.sync_copy(ids_hbm.at[pl.ds(base, per_w)], idx_v)
        pltpu.async_copy(table_hbm.at[idx_v], rows_v, sem).wait()
        pltpu.sync_copy(rows_v, out_hbm.at[pl.ds(base, per_w)])

    return k(table, ids)


def _mm_kernel(x_ref, w_ref, b_ref, o_ref):
    acc = lax.dot_general(
        x_ref[...], w_ref[...], (((1,), (0,)), ((), ())),
        preferred_element_type=jnp.float32)
    o_ref[...] = acc + b_ref[...]


def _mm_bias(x, w, b, bn=256):
    m, k = x.shape
    _, n = w.shape
    return pl.pallas_call(
        _mm_kernel,
        grid=(n // bn,),
        in_specs=[
            pl.BlockSpec((m, k), lambda j: (0, 0)),
            pl.BlockSpec((k, bn), lambda j: (0, j)),
            pl.BlockSpec((1, bn), lambda j: (0, j)),
        ],
        out_specs=pl.BlockSpec((m, bn), lambda j: (0, j)),
        out_shape=jax.ShapeDtypeStruct((m, n), jnp.float32),
        compiler_params=pltpu.CompilerParams(
            dimension_semantics=("arbitrary",)),
    )(x, w, b.reshape(1, n))


def _add_kernel(a_ref, b_ref, o_ref):
    o_ref[...] = a_ref[...] + b_ref[...]


def _add(a, b, bm=256):
    m, n = a.shape
    return pl.pallas_call(
        _add_kernel,
        grid=(m // bm,),
        in_specs=[
            pl.BlockSpec((bm, n), lambda i: (i, 0)),
            pl.BlockSpec((bm, n), lambda i: (i, 0)),
        ],
        out_specs=pl.BlockSpec((bm, n), lambda i: (i, 0)),
        out_shape=jax.ShapeDtypeStruct((m, n), jnp.float32),
    )(a, b)


def _ln_kernel(x_ref, r_ref, g_ref, b_ref, o_ref):
    x = x_ref[...] + r_ref[...]
    mu = jnp.mean(x, axis=-1, keepdims=True)
    xc = x - mu
    var = jnp.mean(xc * xc, axis=-1, keepdims=True)
    o_ref[...] = xc * lax.rsqrt(var + 1e-5) * g_ref[...] + b_ref[...]


def _ln_res(x, r, g, b, bm=256):
    m, n = x.shape
    return pl.pallas_call(
        _ln_kernel,
        grid=(m // bm,),
        in_specs=[
            pl.BlockSpec((bm, n), lambda i: (i, 0)),
            pl.BlockSpec((bm, n), lambda i: (i, 0)),
            pl.BlockSpec((1, n), lambda i: (0, 0)),
            pl.BlockSpec((1, n), lambda i: (0, 0)),
        ],
        out_specs=pl.BlockSpec((bm, n), lambda i: (i, 0)),
        out_shape=jax.ShapeDtypeStruct((m, n), jnp.float32),
    )(x, r, g.reshape(1, n), b.reshape(1, n))


def _attn_kernel(q_ref, k_ref, v_ref, o_ref, *, causal, bq):
    i = pl.program_id(0)
    skv = k_ref.shape[0]
    scale = 1.0 / math.sqrt(DK)
    if not causal:
        for h in range(H):
            sl = pl.ds(h * DK, DK)
            s = lax.dot_general(q_ref[:, sl], k_ref[:, sl],
                                (((1,), (1,)), ((), ())),
                                preferred_element_type=jnp.float32) * scale
            m = jnp.max(s, axis=-1, keepdims=True)
            p = jnp.exp(s - m)
            p = p * (1.0 / jnp.sum(p, axis=-1, keepdims=True))
            o_ref[:, sl] = lax.dot_general(p, v_ref[:, sl],
                                           (((1,), (0,)), ((), ())),
                                           preferred_element_type=jnp.float32)
        return
    def _causal_body(kvlen):
        rows = i * bq + lax.broadcasted_iota(jnp.int32, (bq, kvlen), 0)
        cols = lax.broadcasted_iota(jnp.int32, (bq, kvlen), 1)
        mask = rows >= cols
        for h in range(H):
            sl = pl.ds(h * DK, DK)
            s = lax.dot_general(q_ref[pl.ds(0, bq), sl],
                                k_ref[pl.ds(0, kvlen), sl],
                                (((1,), (1,)), ((), ())),
                                preferred_element_type=jnp.float32) * scale
            s = jnp.where(mask, s, _NEG)
            m = jnp.max(s, axis=-1, keepdims=True)
            p = jnp.exp(s - m)
            p = p * (1.0 / jnp.sum(p, axis=-1, keepdims=True))
            o_ref[pl.ds(0, bq), sl] = lax.dot_general(
                p, v_ref[pl.ds(0, kvlen), sl], (((1,), (0,)), ((), ())),
                preferred_element_type=jnp.float32)

    nblk = skv // bq
    ntiers = 4 if nblk % 4 == 0 else 1
    per = nblk // ntiers
    for t in range(ntiers):
        lo, hi = t * per, (t + 1) * per
        cond = (i >= lo) & (i < hi) if ntiers > 1 else i >= 0

        @pl.when(cond)
        def _(t=t):
            _causal_body(bq * (t + 1) * per)


def _attention(q2, k2, v2, causal, bq=256):
    sq, d = q2.shape
    skv = k2.shape[0]
    return pl.pallas_call(
        functools.partial(_attn_kernel, causal=causal, bq=bq),
        grid=(sq // bq,),
        in_specs=[
            pl.BlockSpec((bq, d), lambda i: (i, 0)),
            pl.BlockSpec((skv, d), lambda i: (0, 0)),
            pl.BlockSpec((skv, d), lambda i: (0, 0)),
        ],
        out_specs=pl.BlockSpec((bq, d), lambda i: (i, 0)),
        out_shape=jax.ShapeDtypeStruct((sq, d), jnp.float32),
        compiler_params=pltpu.CompilerParams(
            dimension_semantics=("arbitrary",)),
    )(q2, k2, v2)


def _router_kernel(x_ref, wr_ref, br_ref, g_ref, c_ref):
    logits = lax.dot_general(x_ref[...], wr_ref[...], (((1,), (0,)), ((), ())),
                             preferred_element_type=jnp.float32) + br_ref[...]
    mx = jnp.max(logits, axis=-1, keepdims=True)
    ex = jnp.exp(logits - mx)
    probs = ex / jnp.sum(ex, axis=-1, keepdims=True)
    iota = lax.broadcasted_iota(jnp.int32, probs.shape, 1)
    m1 = jnp.max(probs, axis=-1, keepdims=True)
    i1 = jnp.min(jnp.where(probs == m1, iota, E), axis=-1, keepdims=True)
    oh1 = iota == i1
    p2 = jnp.where(oh1, -1.0, probs)
    m2 = jnp.max(p2, axis=-1, keepdims=True)
    i2 = jnp.min(jnp.where(p2 == m2, iota, E), axis=-1, keepdims=True)
    oh2 = iota == i2
    denom = m1 + m2
    g_ref[...] = (jnp.where(oh1, m1, 0.0) + jnp.where(oh2, m2, 0.0)) / denom

    blockcnt = jnp.sum(jnp.where(oh1 | oh2, 1.0, 0.0), axis=0, keepdims=True)

    @pl.when(pl.program_id(0) == 0)
    def _():
        c_ref[...] = jnp.zeros_like(c_ref)

    c_ref[...] += blockcnt


def _router(x, wr, br, bm=256):
    m, k = x.shape
    e = wr.shape[1]
    return pl.pallas_call(
        _router_kernel,
        grid=(m // bm,),
        in_specs=[
            pl.BlockSpec((bm, k), lambda i: (i, 0)),
            pl.BlockSpec((k, e), lambda i: (0, 0)),
            pl.BlockSpec((1, e), lambda i: (0, 0)),
        ],
        out_specs=[
            pl.BlockSpec((bm, e), lambda i: (i, 0)),
            pl.BlockSpec((1, e), lambda i: (0, 0)),
        ],
        out_shape=[
            jax.ShapeDtypeStruct((m, e), jnp.float32),
            jax.ShapeDtypeStruct((1, e), jnp.float32),
        ],
        compiler_params=pltpu.CompilerParams(
            dimension_semantics=("arbitrary",)),
    )(x, wr, br.reshape(1, e))


def _plan_kernel(gf_ref, gb_ref, s_ref, *, bm, s_tot):
    i = pl.program_id(0)
    m = jnp.where(gf_ref[...] > 0.0, 1.0, 0.0)
    rows = i * bm + lax.broadcasted_iota(jnp.int32, (bm, s_tot), 0)
    cols = lax.broadcasted_iota(jnp.int32, (bm, s_tot), 1)
    lt = jnp.where(cols < rows, 1.0, 0.0)
    cnt = lax.dot_general(lt, m, (((1,), (0,)), ((), ())),
                          preferred_element_type=jnp.float32)
    g = gb_ref[...]
    iota = lax.broadcasted_iota(jnp.int32, g.shape, 1)
    sel = g > 0.0
    e1 = jnp.min(jnp.where(sel, iota, E), axis=-1, keepdims=True)
    e1 = jnp.minimum(e1, E - 1)
    e2 = jnp.max(jnp.where(sel, iota, 0), axis=-1, keepdims=True)
    rank1 = jnp.sum(jnp.where(iota == e1, cnt, 0.0), axis=-1, keepdims=True)
    rank2 = jnp.sum(jnp.where(iota == e2, cnt, 0.0), axis=-1, keepdims=True)
    slot1 = e1 * s_tot + rank1.astype(jnp.int32)
    slot2 = e2 * s_tot + rank2.astype(jnp.int32)
    s_ref[...] = (jnp.where(iota == 0, slot1, 0)
                  + jnp.where(iota == 1, slot2, 0))


def _route_plan(gates, bm=256):
    m, e = gates.shape
    return pl.pallas_call(
        functools.partial(_plan_kernel, bm=bm, s_tot=m),
        grid=(m // bm,),
        in_specs=[
            pl.BlockSpec((m, e), lambda i: (0, 0)),
            pl.BlockSpec((bm, e), lambda i: (i, 0)),
        ],
        out_specs=pl.BlockSpec((bm, e), lambda i: (i, 0)),
        out_shape=jax.ShapeDtypeStruct((m, e), jnp.int32),
    )(gates, gates)


def _sc_dispatch(x, i1, i2):
    nw = 32
    per_w = S // nw
    mesh = plsc.VectorSubcoreMesh(core_axis_name="c", subcore_axis_name="s")

    @functools.partial(
        pl.kernel,
        mesh=mesh,
        out_type=jax.ShapeDtypeStruct((E * S, D), jnp.float32),
        scratch_types=[
            pltpu.VMEM((per_w,), jnp.int32),
            pltpu.VMEM((per_w,), jnp.int32),
            pltpu.VMEM((per_w, D), jnp.float32),
            pltpu.SemaphoreType.DMA,
            pltpu.SemaphoreType.DMA,
        ],
    )
    def k(x_hbm, i1_hbm, i2_hbm, xg_hbm, i1_v, i2_v, rows_v, sem1, sem2):
        wid = lax.axis_index("s") * 2 + lax.axis_index("c")
        base = wid * per_w
        pltpu.sync_copy(x_hbm.at[pl.ds(base, per_w)], rows_v)
        pltpu.sync_copy(i1_hbm.at[pl.ds(base, per_w)], i1_v)
        pltpu.sync_copy(i2_hbm.at[pl.ds(base, per_w)], i2_v)
        c1 = pltpu.async_copy(rows_v, xg_hbm.at[i1_v], sem1)
        c2 = pltpu.async_copy(rows_v, xg_hbm.at[i2_v], sem2)
        c1.wait()
        c2.wait()

    return k(x, i1, i2)


def _sc_combine_gather(yg, i1, i2):
    nw = 32
    per_w = S // nw
    mesh = plsc.VectorSubcoreMesh(core_axis_name="c", subcore_axis_name="s")

    @functools.partial(
        pl.kernel,
        mesh=mesh,
        out_type=[
            jax.ShapeDtypeStruct((S, D), jnp.float32),
            jax.ShapeDtypeStruct((S, D), jnp.float32),
        ],
        scratch_types=[
            pltpu.VMEM((per_w,), jnp.int32),
            pltpu.VMEM((per_w,), jnp.int32),
            pltpu.VMEM((per_w, D), jnp.float32),
            pltpu.VMEM((per_w, D), jnp.float32),
            pltpu.SemaphoreType.DMA,
            pltpu.SemaphoreType.DMA,
        ],
    )
    def k(yg_hbm, i1_hbm, i2_hbm, y1_hbm, y2_hbm,
          i1_v, i2_v, r1_v, r2_v, sem1, sem2):
        wid = lax.axis_index("s") * 2 + lax.axis_index("c")
        base = wid * per_w
        pltpu.sync_copy(i1_hbm.at[pl.ds(base, per_w)], i1_v)
        pltpu.sync_copy(i2_hbm.at[pl.ds(base, per_w)], i2_v)
        c1 = pltpu.async_copy(yg_hbm.at[i1_v], r1_v, sem1)
        c2 = pltpu.async_copy(yg_hbm.at[i2_v], r2_v, sem2)
        c1.wait()
        c2.wait()
        pltpu.sync_copy(r1_v, y1_hbm.at[pl.ds(base, per_w)])
        pltpu.sync_copy(r2_v, y2_hbm.at[pl.ds(base, per_w)])

    return k(yg, i1, i2)


def _ffn_kernel(cnt_ref, xg_ref, w1_ref, b1_ref, w2_ref, b2_ref, o_ref, *, bm):
    e = pl.program_id(0)
    j = pl.program_id(1)

    @pl.when(j * bm < cnt_ref[e])
    def _():
        h1 = lax.dot_general(xg_ref[0], w1_ref[0], (((1,), (0,)), ((), ())),
                             preferred_element_type=jnp.float32) + b1_ref[0]
        h1 = jnp.maximum(h1, 0.0)
        o_ref[0] = lax.dot_general(h1, w2_ref[0], (((1,), (0,)), ((), ())),
                                   preferred_element_type=jnp.float32) + b2_ref[0]


def _ffn_sparse(xg3, counts, w1, b1, w2, b2, bm=256):
    e, s, d = xg3.shape
    dff = w1.shape[2]

    def xg_map(ei, j, cnt):
        nb = jnp.maximum(pl.cdiv(cnt[ei], bm) - 1, 0)
        return (ei, jnp.minimum(j, nb), 0)

    grid_spec = pltpu.PrefetchScalarGridSpec(
        num_scalar_prefetch=1,
        grid=(e, s // bm),
        in_specs=[
            pl.BlockSpec((1, bm, d), xg_map),
            pl.BlockSpec((1, d, dff), lambda ei, j, cnt: (ei, 0, 0)),
            pl.BlockSpec((1, 1, dff), lambda ei, j, cnt: (ei, 0, 0)),
            pl.BlockSpec((1, dff, d), lambda ei, j, cnt: (ei, 0, 0)),
            pl.BlockSpec((1, 1, d), lambda ei, j, cnt: (ei, 0, 0)),
        ],
        out_specs=pl.BlockSpec((1, bm, d), xg_map),
    )
    return pl.pallas_call(
        functools.partial(_ffn_kernel, bm=bm),
        grid_spec=grid_spec,
        out_shape=jax.ShapeDtypeStruct((e, s, d), jnp.float32),
        compiler_params=pltpu.CompilerParams(
            dimension_semantics=("arbitrary", "arbitrary")),
    )(counts, xg3, w1, b1.reshape(e, 1, dff), w2, b2.reshape(e, 1, d))


def _combine_ln_kernel(x_ref, y1_ref, y2_ref, g_ref, gam_ref, bet_ref, o_ref):
    g = g_ref[...]
    iota = lax.broadcasted_iota(jnp.int32, g.shape, 1)
    sel = g > 0.0
    e1 = jnp.min(jnp.where(sel, iota, E), axis=-1, keepdims=True)
    e1 = jnp.minimum(e1, E - 1)
    e2 = jnp.max(jnp.where(sel, iota, 0), axis=-1, keepdims=True)
    w1 = jnp.sum(jnp.where(iota == e1, g, 0.0), axis=-1, keepdims=True)
    w2 = jnp.sum(jnp.where(iota == e2, g, 0.0), axis=-1, keepdims=True)
    w2 = jnp.where(e2 == e1, 0.0, w2)
    x = x_ref[...] + w1 * y1_ref[...] + w2 * y2_ref[...]
    mu = jnp.mean(x, axis=-1, keepdims=True)
    xc = x - mu
    var = jnp.mean(xc * xc, axis=-1, keepdims=True)
    o_ref[...] = xc * lax.rsqrt(var + 1e-5) * gam_ref[...] + bet_ref[...]


def _combine_ln(x, y1, y2, gates, g, b, bm=256):
    m, n = x.shape
    e = gates.shape[1]
    return pl.pallas_call(
        _combine_ln_kernel,
        grid=(m // bm,),
        in_specs=[
            pl.BlockSpec((bm, n), lambda i: (i, 0)),
            pl.BlockSpec((bm, n), lambda i: (i, 0)),
            pl.BlockSpec((bm, n), lambda i: (i, 0)),
            pl.BlockSpec((bm, e), lambda i: (i, 0)),
            pl.BlockSpec((1, n), lambda i: (0, 0)),
            pl.BlockSpec((1, n), lambda i: (0, 0)),
        ],
        out_specs=pl.BlockSpec((bm, n), lambda i: (i, 0)),
        out_shape=jax.ShapeDtypeStruct((m, n), jnp.float32),
    )(x, y1, y2, gates, g.reshape(1, n), b.reshape(1, n))


def _mha(xq, xkv, p, causal=False):
    wqkv = jnp.concatenate([p['wq'], p['wk'], p['wv']], axis=1)
    bqkv = jnp.concatenate([p['bq'], p['bk'], p['bv']], axis=0)
    if xq is xkv:
        qkv = _mm_bias(xq, wqkv, bqkv, bn=768)
        q, k, v = qkv[:, :D], qkv[:, D:2 * D], qkv[:, 2 * D:]
    else:
        q = _mm_bias(xq, p['wq'], p['bq'])
        wkv = jnp.concatenate([p['wk'], p['wv']], axis=1)
        bkv = jnp.concatenate([p['bk'], p['bv']], axis=0)
        kv = _mm_bias(xkv, wkv, bkv, bn=512)
        k, v = kv[:, :D], kv[:, D:]
    ctx = _attention(q, k, v, causal)
    return _mm_bias(ctx, p['wo'], p['bo'])


def _moe_ln_block(x, p, lnp):
    gates, counts = _router(x, p['wr'], p['br'])
    slotpack = _route_plan(gates)
    i1 = slotpack[:, 0]
    i2 = slotpack[:, 1]
    xg = _sc_dispatch(x, i1, i2)
    cnt_i = counts.reshape(E).astype(jnp.int32)
    yg = _ffn_sparse(xg.reshape(E, S, D), cnt_i,
                     p['w1'], p['b1'], p['w2'], p['b2'])
    y1, y2 = _sc_combine_gather(yg.reshape(E * S, D), i1, i2)
    return _combine_ln(x, y1, y2, gates, lnp['g'], lnp['b'])


def kernel(src_ids, tgt_ids, params):
    src = src_ids.reshape(-1).astype(jnp.int32)
    tgt = tgt_ids.reshape(-1).astype(jnp.int32)
    ids = jnp.concatenate([src, tgt], axis=0)
    rows = _sc_embed_gather(params['emb'], ids, 2 * S)
    pe = jnp.asarray(_pe_np(S, D))
    x = _add(rows[:S], pe)
    y0 = _add(rows[S:], pe)

    lp = params['enc'][0]
    a = _mha(x, x, lp['attn'])
    x = _ln_res(x, a, lp['ln1']['g'], lp['ln1']['b'])
    x = _moe_ln_block(x, lp['moe'], lp['ln2'])
    enc_out = x

    lp = params['dec'][0]
    y = y0
    a = _mha(y, y, lp['sattn'], causal=True)
    y = _ln_res(y, a, lp['ln1']['g'], lp['ln1']['b'])
    c = _mha(y, enc_out, lp['cattn'])
    y = _ln_res(y, c, lp['ln2']['g'], lp['ln2']['b'])
    y = _moe_ln_block(y, lp['moe'], lp['ln3'])

    logits = _mm_bias(y, params['wout'], params['bout'], bn=1024)
    return logits.reshape(1, S, V)
